# baseline (speedup 1.0000x reference)
import jax
import jax.numpy as jnp
from jax import lax
from jax.experimental import pallas as pl
from jax.experimental.pallas import tpu as pltpu
from jax.experimental.pallas import tpu_sc as plsc

ENT_TOTAL = 100000
REL_TOTAL = 1000
HIDDEN = 64
HALF = HIDDEN // 2
BATCH = 16384
LMBDA = 0.0001

NC = 2
NS = 16
LANES = 16
NW = NC * NS
ROWS_PER_W = BATCH // NW
CHUNK = 64
NCHUNK = ROWS_PER_W // CHUNK
NBUF = 2
NIDX = 9


def _row_block(offs, bufs, i, k, acc32, acc64):
    e1h_b, e2h_b, eh_b, e1t_b, e2t_b, et_b, rel_b = bufs
    o32h_v, o64h_v, o32t_v, o64t_v = offs
    o32h = pl.multiple_of(o32h_v[k], HALF)
    o64h = pl.multiple_of(o64h_v[k], HIDDEN)
    o32t = pl.multiple_of(o32t_v[k], HALF)
    o64t = pl.multiple_of(o64t_v[k], HIDDEN)
    comp = jnp.zeros((LANES,), jnp.float32)
    dist = jnp.zeros((LANES,), jnp.float32)
    for c in range(0, HALF, LANES):
        a1 = e1h_b[i, pl.ds(o32h + c, LANES)]
        a2 = e2h_b[i, pl.ds(o32h + c, LANES)]
        b1 = e1t_b[i, pl.ds(o32t + c, LANES)]
        b2 = e2t_b[i, pl.ds(o32t + c, LANES)]
        q1 = rel_b[i, pl.ds(c, LANES)]
        q2 = rel_b[i, pl.ds(HALF + c, LANES)]
        comp = comp + (a1 * b1 + a2 * b2) * q1 + (a1 * b2 - a2 * b1) * q2
        acc32 = acc32 + a1 * a1 + a2 * a2 + b1 * b1 + b2 * b2 + q1 * q1 + q2 * q2
    for c in range(0, HIDDEN, LANES):
        x = eh_b[i, pl.ds(o64h + c, LANES)]
        z = et_b[i, pl.ds(o64t + c, LANES)]
        w = rel_b[i, pl.ds(HIDDEN + c, LANES)]
        dist = dist + x * z * w
        acc64 = acc64 + x * x + z * z + w * w
    total = jnp.sum(comp + dist)
    return total, acc32, acc64


def _sc_body(idx_hbm, e1_hbm, e2_hbm, e_hbm, rel_hbm,
             res_hbm, p32_hbm, p64_hbm,
             slots, res_v, p32_v, p64_v, sems):
    wid = lax.axis_index("s") * NC + lax.axis_index("c")
    lane = lax.iota(jnp.int32, LANES)

    def fire(g, s):
        idx_b, bufs = slots[s]
        pltpu.sync_copy(idx_hbm.at[wid, g], idx_b)
        tables = (e1_hbm, e2_hbm, e_hbm, e1_hbm, e2_hbm, e_hbm, rel_hbm)
        rows = (0, 0, 2, 4, 4, 6, 8)
        return [pltpu.async_copy(tab.at[idx_b.at[k]], buf, sems[s])
                for tab, k, buf in zip(tables, rows, bufs)]

    pending = {0: fire(0, 0)}
    for g in range(NCHUNK):
        s = g % NBUF
        if g + 1 < NCHUNK:
            pending[g + 1] = fire(g + 1, (g + 1) % NBUF)
        for d in pending.pop(g):
            d.wait()
        idx_b, bufs = slots[s]

        def body(i16, carry, _idx=idx_b, _bufs=bufs, _g=g):
            acc32, acc64 = carry
            res_vec = jnp.zeros((LANES,), jnp.float32)
            offs = tuple(_idx[j, pl.ds(i16 * LANES, LANES)]
                         for j in (1, 3, 5, 7))
            for k in range(LANES):
                total, acc32, acc64 = _row_block(offs, _bufs,
                                                 i16 * LANES + k, k,
                                                 acc32, acc64)
                res_vec = jnp.where(lane == k, total, res_vec)
            res_v[pl.ds(_g * CHUNK + i16 * LANES, LANES)] = res_vec
            return acc32, acc64

        if g == 0:
            carry = (jnp.zeros((LANES,), jnp.float32),
                     jnp.zeros((LANES,), jnp.float32))
        carry = lax.fori_loop(0, CHUNK // LANES, body, carry)

    acc32, acc64 = carry
    p32_v[...] = acc32
    p64_v[...] = acc64
    pltpu.sync_copy(res_v, res_hbm.at[pl.ds(wid * ROWS_PER_W, ROWS_PER_W)])
    pltpu.sync_copy(p32_v, p32_hbm.at[wid])
    pltpu.sync_copy(p64_v, p64_hbm.at[wid])


def _make_sc_call():
    mesh = plsc.VectorSubcoreMesh(core_axis_name="c", subcore_axis_name="s")
    slot = lambda: (pltpu.VMEM((NIDX, CHUNK), jnp.int32),
                    tuple(pltpu.VMEM((CHUNK, 128), jnp.float32)
                          for _ in range(7)))
    return pl.kernel(
        _sc_body,
        out_type=(jax.ShapeDtypeStruct((BATCH,), jnp.float32),
                  jax.ShapeDtypeStruct((NW, LANES), jnp.float32),
                  jax.ShapeDtypeStruct((NW, LANES), jnp.float32)),
        mesh=mesh,
        compiler_params=pltpu.CompilerParams(needs_layout_passes=False,
                                             use_tc_tiling_on_sc=False),
        scratch_types=[
            tuple(slot() for _ in range(NBUF)),
            pltpu.VMEM((ROWS_PER_W,), jnp.float32),
            pltpu.VMEM((LANES,), jnp.float32),
            pltpu.VMEM((LANES,), jnp.float32),
            tuple(pltpu.SemaphoreType.DMA for _ in range(NBUF)),
        ],
    )


def _finish_body(res_ref, y_ref, p32_ref, p64_ref, out_ref):
    z = -y_ref[...] * res_ref[...]
    sp = jnp.maximum(z, 0.0) + jnp.log1p(jnp.exp(-jnp.abs(z)))
    loss = jnp.sum(sp) * (1.0 / BATCH)
    regul = (jnp.sum(p32_ref[...]) * (1.0 / (BATCH * HALF))
             + jnp.sum(p64_ref[...]) * (1.0 / (BATCH * HIDDEN)))
    out_ref[0, 0] = loss + LMBDA * regul


def kernel(h, t, r, y, ent1_embeddings, ent2_embeddings, ent_embeddings,
           rel1_embeddings, rel2_embeddings, rel_embeddings):
    h = h.astype(jnp.int32)
    t = t.astype(jnp.int32)
    r = r.astype(jnp.int32)
    idx = jnp.stack([
        h >> 2, (h & 3) * HALF, h >> 1, (h & 1) * HIDDEN,
        t >> 2, (t & 3) * HALF, t >> 1, (t & 1) * HIDDEN,
        r,
    ])
    idx = (idx.reshape(NIDX, NW, NCHUNK, CHUNK)
           .transpose(1, 2, 0, 3))
    e1r = ent1_embeddings.reshape(ENT_TOTAL // 4, 128)
    e2r = ent2_embeddings.reshape(ENT_TOTAL // 4, 128)
    er = ent_embeddings.reshape(ENT_TOTAL // 2, 128)
    relcat = jnp.concatenate(
        [rel1_embeddings, rel2_embeddings, rel_embeddings], axis=1)
    sc = _make_sc_call()
    res, p32, p64 = sc(idx, e1r, e2r, er, relcat)
    out = pl.pallas_call(
        _finish_body,
        out_shape=jax.ShapeDtypeStruct((1, 1), jnp.float32),
        out_specs=pl.BlockSpec(memory_space=pltpu.SMEM),
    )(res.reshape(128, 128), y.reshape(128, 128),
      p32.reshape(4, 128), p64.reshape(4, 128))
    return out[0, 0]
